# baseline (speedup 1.0000x reference)
import functools

import jax
import jax.numpy as jnp
from jax import lax
from jax.experimental import pallas as pl
from jax.experimental.pallas import tpu as pltpu
from jax.experimental.pallas import tpu_sc as plsc

DIM = 64
NC, NS, L = 2, 16, 16
NW = NC * NS
NBUF = 4
NOB = 2
LA = 2


@functools.lru_cache(maxsize=None)
def _make_sc_gather(batch: int, hist: int, vocab: int):
    assert batch % NW == 0
    bw = batch // NW
    nbg = bw // L
    mesh = plsc.VectorSubcoreMesh(core_axis_name="c", subcore_axis_name="s")

    @functools.partial(
        pl.kernel,
        mesh=mesh,
        compiler_params=pltpu.CompilerParams(needs_layout_passes=False),
        out_type=jax.ShapeDtypeStruct((hist, DIM, batch), jnp.float32),
        scratch_types=[
            pltpu.VMEM((hist, bw), jnp.int32),
            pltpu.VMEM((NBUF, bw), jnp.int32),
            pltpu.VMEM((NBUF, bw), jnp.int32),
            pltpu.VMEM((NBUF, bw, 2 * DIM), jnp.float32),
            pltpu.VMEM((NOB, DIM, bw), jnp.float32),
        ]
        + [pltpu.SemaphoreType.DMA] * NBUF
        + [pltpu.SemaphoreType.DMA] * NOB,
    )
    def k(idx_hbm, tab2_hbm, out_hbm, idx_v, pix_v, off_v, rows_v, outv, *sems):
        gsem = sems[:NBUF]
        wsem = sems[NBUF:]
        wid = lax.axis_index("s") * NC + lax.axis_index("c")
        base = wid * bw
        pltpu.sync_copy(idx_hbm.at[:, pl.ds(base, bw)], idx_v)

        def prep(h, i):
            for g in range(nbg):
                x = idx_v[h, pl.ds(g * L, L)]
                pix_v[i, pl.ds(g * L, L)] = lax.shift_right_logical(x, 1)
                off_v[i, pl.ds(g * L, L)] = lax.mul(
                    lax.bitwise_and(x, 1), jnp.int32(DIM)
                )

        def gather(h, i):
            del h
            pltpu.async_copy(tab2_hbm.at[pix_v.at[i]], rows_v.at[i], gsem[i])

        for h in range(LA):
            prep(h, h)
            gather(h, h)

        def slot(h, i, o, first, last):
            pltpu.make_async_copy(
                tab2_hbm.at[pix_v.at[i]], rows_v.at[i], gsem[i]
            ).wait()
            if not last:
                g2 = h + LA
                j = (i + LA) % NBUF
                prep(g2, j)
                gather(g2, j)
            if not first:
                pltpu.make_async_copy(
                    outv.at[o], out_hbm.at[0, pl.ds(0, DIM), pl.ds(base, bw)],
                    wsem[o],
                ).wait()
            for g in range(nbg):
                bvec = jax.lax.iota(jnp.int32, L) + jnp.int32(g * L)
                offs = off_v[i, pl.ds(g * L, L)]

                def dbody(d, carry):
                    v = plsc.load_gather(rows_v.at[i], [bvec, offs + d])
                    outv[o, d, pl.ds(g * L, L)] = v
                    return carry

                lax.fori_loop(0, DIM, dbody, 0)
            pltpu.async_copy(
                outv.at[o], out_hbm.at[h, pl.ds(0, DIM), pl.ds(base, bw)],
                wsem[o],
            )

        n_main = hist - LA
        assert n_main % NBUF == 0

        def outer(kk, carry):
            h0 = kk * NBUF
            for s in range(NBUF):
                slot(h0 + s, s, s % NOB, first=False, last=False)
            return carry

        for s in range(NBUF):
            slot(s, s, s % NOB, first=(s < NOB), last=False)
        lax.fori_loop(1, n_main // NBUF, outer, 0)
        for t in range(LA):
            h = n_main + t
            slot(h, h % NBUF, h % NOB, first=False, last=True)

        for t in range(NOB):
            o = (hist - 1 - t) % NOB
            pltpu.make_async_copy(
                outv.at[o], out_hbm.at[0, pl.ds(0, DIM), pl.ds(base, bw)],
                wsem[o],
            ).wait()

    return k


def kernel(inputs, table):
    batch, hist = inputs.shape
    vocab = table.shape[0]
    table2 = table.reshape(vocab // 2, 2 * DIM)
    out = _make_sc_gather(batch, hist, vocab)(inputs.T, table2)
    return jnp.transpose(out, (2, 0, 1))


# speedup vs baseline: 1.1415x; 1.1415x over previous
import functools

import jax
import jax.numpy as jnp
from jax import lax
from jax.experimental import pallas as pl
from jax.experimental.pallas import tpu as pltpu
from jax.experimental.pallas import tpu_sc as plsc

DIM = 64
NC, NS, L = 2, 16, 16
NW = NC * NS
NBUF = 4
NOB = 2
LA = 2


@functools.lru_cache(maxsize=None)
def _make_sc_gather(batch: int, hist: int, vocab: int):
    assert batch % NW == 0
    bw = batch // NW
    nbg = bw // L
    mesh = plsc.VectorSubcoreMesh(core_axis_name="c", subcore_axis_name="s")

    @functools.partial(
        pl.kernel,
        mesh=mesh,
        compiler_params=pltpu.CompilerParams(needs_layout_passes=False),
        out_type=jax.ShapeDtypeStruct((batch, hist, DIM), jnp.float32),
        scratch_types=[
            pltpu.VMEM((hist, bw), jnp.int32),
            pltpu.VMEM((NBUF, bw), jnp.int32),
            pltpu.VMEM((NBUF, bw), jnp.int32),
            pltpu.VMEM((NBUF, bw, 2 * DIM), jnp.float32),
            pltpu.VMEM((NOB, bw, DIM), jnp.float32),
        ]
        + [pltpu.SemaphoreType.DMA] * NBUF
        + [pltpu.SemaphoreType.DMA] * NOB,
    )
    def k(idx_hbm, tab2_hbm, out_hbm, idx_v, pix_v, off_v, rows_v, outv, *sems):
        gsem = sems[:NBUF]
        wsem = sems[NBUF:]
        wid = lax.axis_index("s") * NC + lax.axis_index("c")
        base = wid * bw
        pltpu.sync_copy(idx_hbm.at[:, pl.ds(base, bw)], idx_v)

        def prep(h, i):
            for g in range(nbg):
                x = idx_v[h, pl.ds(g * L, L)]
                pix_v[i, pl.ds(g * L, L)] = lax.shift_right_logical(x, 1)
                off_v[i, pl.ds(g * L, L)] = lax.mul(
                    lax.bitwise_and(x, 1), jnp.int32(DIM)
                )

        def gather(i):
            pltpu.async_copy(tab2_hbm.at[pix_v.at[i]], rows_v.at[i], gsem[i])

        for h in range(LA):
            prep(h, h)
            gather(h)

        def slot(h, i, o, first, last):
            pltpu.make_async_copy(
                tab2_hbm.at[pix_v.at[i]], rows_v.at[i], gsem[i]
            ).wait()
            if not last:
                j = (i + LA) % NBUF
                prep(h + LA, j)
                gather(j)
            if not first:
                pltpu.make_async_copy(
                    outv.at[o], out_hbm.at[pl.ds(base, bw), 0], wsem[o]
                ).wait()
            def bbody(bg, carry):
                offv = off_v[i, pl.ds(bg * L, L)]
                for u in range(L):
                    b = bg * L + u
                    off = offv[u]
                    for q in range(DIM // L):
                        outv[o, b, pl.ds(q * L, L)] = rows_v[
                            i, b, pl.ds(off + q * L, L)
                        ]
                return carry

            lax.fori_loop(0, bw // L, bbody, 0)
            pltpu.async_copy(
                outv.at[o], out_hbm.at[pl.ds(base, bw), h], wsem[o]
            )

        n_main = hist - LA
        assert n_main % NBUF == 0

        def outer(kk, carry):
            h0 = kk * NBUF
            for s in range(NBUF):
                slot(h0 + s, s, s % NOB, first=False, last=False)
            return carry

        for s in range(NBUF):
            slot(s, s, s % NOB, first=(s < NOB), last=False)
        lax.fori_loop(1, n_main // NBUF, outer, 0)
        for t in range(LA):
            h = n_main + t
            slot(h, h % NBUF, h % NOB, first=False, last=True)

        for t in range(NOB):
            o = (hist - 1 - t) % NOB
            pltpu.make_async_copy(
                outv.at[o], out_hbm.at[pl.ds(base, bw), 0], wsem[o]
            ).wait()

    return k


def kernel(inputs, table):
    batch, hist = inputs.shape
    vocab = table.shape[0]
    table2 = table.reshape(vocab // 2, 2 * DIM)
    return _make_sc_gather(batch, hist, vocab)(inputs.T, table2)
